# baseline (speedup 1.0000x reference)
import functools

import jax
import jax.numpy as jnp
from jax import lax
from jax.experimental import pallas as pl
from jax.experimental.pallas import tpu as pltpu
from jax.experimental.pallas import tpu_sc as plsc

DEP = 128
SLOT = 128
GROUPS = 2
GROUP_SIZE = 1024
TOTAL = GROUPS * GROUP_SIZE

NC = 2
NS = 16
NW = NC * NS
D_PER_W = DEP // NW

CH = 256
NBUF = 3
M_SC = 1
SC_ITEMS = M_SC * CH
NB16_SC = SC_ITEMS // 16

IB_TC = 128
NIB_TC = (GROUP_SIZE - SC_ITEMS) // IB_TC


def _bcast_idx(i):
    return jnp.full((16,), i, jnp.int32)


_GDN = lax.GatherDimensionNumbers(
    offset_dims=(), collapsed_slice_dims=(0,), start_index_map=(0,))


def _lane_bcast(v16, l):
    return lax.gather(v16, _bcast_idx(l)[:, None], _GDN, slice_sizes=(1,),
                      mode=lax.GatherScatterMode.PROMISE_IN_BOUNDS)


_SCHED = [(dl, g, c)
          for dl in range(D_PER_W) for g in range(GROUPS) for c in range(M_SC)]
_NCH = len(_SCHED)


def _sc_body(addr_hbm, embt_hbm, freq_hbm, out_hbm,
             freq_v, fd_v, et_v, abuf, ov, sem0, sem1, sem2):
    wid = lax.axis_index("s") * NC + lax.axis_index("c")
    dd0 = wid * D_PER_W
    sems = (sem0, sem1, sem2)

    def chunk_copy(gc):
        dl, g, c = _SCHED[gc]
        return pltpu.make_async_copy(
            addr_hbm.at[dd0 + dl, pl.ds(g * GROUP_SIZE + c * CH, CH)],
            abuf.at[pl.ds((gc % NBUF) * CH, CH)],
            sems[gc % NBUF],
        )

    pltpu.sync_copy(freq_hbm, freq_v)
    chunk_copy(0).start()
    chunk_copy(1).start()

    for dl in range(D_PER_W):
        pltpu.sync_copy(embt_hbm.at[dd0 + dl], et_v)

        def fd_body(t, _):
            o = t * 16
            fd_v[pl.ds(o, 16)] = et_v[pl.ds(o, 16)] * freq_v[pl.ds(o, 16)]
            return 0

        lax.fori_loop(0, TOTAL // 16, fd_body, 0)

        for g in range(GROUPS):
            gc0 = (dl * GROUPS + g) * M_SC

            def blk_body(i16, acc, gc0=gc0, g=g):
                for c in range(M_SC):
                    @pl.when(i16 == (CH // 16) * c)
                    def _boundary(c=c):
                        if gc0 + c + 2 < _NCH:
                            chunk_copy(gc0 + c + 2).start()
                        chunk_copy(gc0 + c).wait()

                buf = lax.rem(gc0 + lax.div(i16, CH // 16), NBUF)
                r0 = buf * CH + lax.rem(i16, CH // 16) * 16
                f16 = fd_v[pl.ds(g * GROUP_SIZE + i16 * 16, 16)]
                acc = list(acc)
                for l in range(16):
                    fbc = _lane_bcast(f16, l)
                    for j in range(8):
                        acc[j] = acc[j] + fbc * abuf[r0 + l, pl.ds(j * 16, 16)]
                return tuple(acc)

            zero = jnp.zeros((16,), jnp.float32)
            acc = lax.fori_loop(0, NB16_SC, blk_body, (zero,) * 8)
            for j in range(8):
                ov[g, dl, pl.ds(j * 16, 16)] = acc[j]

    for g in range(GROUPS):
        pltpu.sync_copy(ov.at[g], out_hbm.at[g, pl.ds(dd0, D_PER_W)])


def _sc_part(batch_address, embt, batch_frequency):
    mesh = plsc.VectorSubcoreMesh(core_axis_name="c", subcore_axis_name="s")
    f = functools.partial(
        pl.kernel,
        mesh=mesh,
        out_type=jax.ShapeDtypeStruct((GROUPS, DEP, SLOT), jnp.float32),
        scratch_types=[
            pltpu.VMEM((TOTAL,), jnp.float32),
            pltpu.VMEM((TOTAL,), jnp.float32),
            pltpu.VMEM((TOTAL,), jnp.float32),
            pltpu.VMEM((NBUF * CH, SLOT), jnp.float32),
            pltpu.VMEM((GROUPS, D_PER_W, SLOT), jnp.float32),
            pltpu.SemaphoreType.DMA,
            pltpu.SemaphoreType.DMA,
            pltpu.SemaphoreType.DMA,
        ],
    )(_sc_body)
    return f(batch_address, embt, batch_frequency)


def _tc_body(addr_ref, embt_ref, freq_ref, mem_ref, out_ref):
    ib = pl.program_id(1)
    a = addr_ref[...]
    ft = embt_ref[...] * freq_ref[...]
    contrib = jnp.sum(a * ft[:, :, None], axis=1)

    @pl.when(ib == 0)
    def _init():
        out_ref[...] = mem_ref[...] + contrib[None]

    @pl.when(ib != 0)
    def _acc():
        out_ref[...] += contrib[None]


def _tc_part(batch_address, embt, freq2d, memory_matrix):
    nb_per_group = GROUP_SIZE // IB_TC
    grid = (GROUPS, NIB_TC)

    off = SC_ITEMS // IB_TC

    def imap(g, ib):
        return (0, g * nb_per_group + off + ib, 0)

    return pl.pallas_call(
        _tc_body,
        grid=grid,
        in_specs=[
            pl.BlockSpec((DEP, IB_TC, SLOT), imap),
            pl.BlockSpec((DEP, IB_TC), lambda g, ib: (0, g * nb_per_group + off + ib)),
            pl.BlockSpec((1, IB_TC), lambda g, ib: (0, g * nb_per_group + off + ib)),
            pl.BlockSpec((1, DEP, SLOT), lambda g, ib: (g, 0, 0)),
        ],
        out_specs=pl.BlockSpec((1, DEP, SLOT), lambda g, ib: (g, 0, 0)),
        out_shape=jax.ShapeDtypeStruct((GROUPS, DEP, SLOT), jnp.float32),
        compiler_params=pltpu.CompilerParams(
            dimension_semantics=("arbitrary", "arbitrary"),
            skip_device_barrier=True,
        ),
    )(batch_address, embt, freq2d, memory_matrix)


def kernel(batch_address, batch_embedding, batch_frequency, memory_matrix):
    embt = batch_embedding.T
    sc_out = _sc_part(batch_address, embt, batch_frequency)
    tc_out = _tc_part(batch_address, embt, batch_frequency[None, :], memory_matrix)
    return sc_out + tc_out


# speedup vs baseline: 1.1053x; 1.1053x over previous
import jax
import jax.numpy as jnp
from jax.experimental import pallas as pl
from jax.experimental.pallas import tpu as pltpu

DEP = 128
SLOT = 128
GROUPS = 2
GROUP_SIZE = 1024
DB = 8


def _body(addr_ref, embt_ref, freq_ref, mem_ref, out_ref):
    a = addr_ref[...]
    ft = embt_ref[...] * freq_ref[...]
    contrib = jnp.sum(a * ft[:, :, None], axis=1)
    out_ref[...] = mem_ref[...] + contrib[None]


def kernel(batch_address, batch_embedding, batch_frequency, memory_matrix):
    embt = batch_embedding.T
    freq = batch_frequency[None, :]
    n_db = DEP // DB
    grid = (GROUPS, n_db)
    return pl.pallas_call(
        _body,
        grid=grid,
        in_specs=[
            pl.BlockSpec((DB, GROUP_SIZE, SLOT), lambda g, db: (db, g, 0)),
            pl.BlockSpec((DB, GROUP_SIZE), lambda g, db: (db, g)),
            pl.BlockSpec((1, GROUP_SIZE), lambda g, db: (0, g)),
            pl.BlockSpec((1, DB, SLOT), lambda g, db: (g, db, 0)),
        ],
        out_specs=pl.BlockSpec((1, DB, SLOT), lambda g, db: (g, db, 0)),
        out_shape=jax.ShapeDtypeStruct((GROUPS, DEP, SLOT), jnp.float32),
        compiler_params=pltpu.CompilerParams(
            dimension_semantics=("arbitrary", "arbitrary"),
        ),
    )(batch_address, embt, freq, memory_matrix)


# speedup vs baseline: 1.3542x; 1.2252x over previous
import jax
import jax.numpy as jnp
from jax.experimental import pallas as pl
from jax.experimental.pallas import tpu as pltpu

DEP = 128
SLOT = 128
GROUPS = 2
GROUP_SIZE = 1024
DB = 32


def _body(addr_ref, embt_ref, freq_ref, mem_ref, out_ref):
    a = addr_ref[...]
    ft = embt_ref[...] * freq_ref[...]
    contrib = jnp.sum(a * ft[:, :, None], axis=1)
    out_ref[...] = mem_ref[...] + contrib[None]


def kernel(batch_address, batch_embedding, batch_frequency, memory_matrix):
    embt = batch_embedding.T
    freq = batch_frequency[None, :]
    n_db = DEP // DB
    grid = (GROUPS, n_db)
    return pl.pallas_call(
        _body,
        grid=grid,
        in_specs=[
            pl.BlockSpec((DB, GROUP_SIZE, SLOT), lambda g, db: (db, g, 0)),
            pl.BlockSpec((DB, GROUP_SIZE), lambda g, db: (db, g)),
            pl.BlockSpec((1, GROUP_SIZE), lambda g, db: (0, g)),
            pl.BlockSpec((1, DB, SLOT), lambda g, db: (g, db, 0)),
        ],
        out_specs=pl.BlockSpec((1, DB, SLOT), lambda g, db: (g, db, 0)),
        out_shape=jax.ShapeDtypeStruct((GROUPS, DEP, SLOT), jnp.float32),
        compiler_params=pltpu.CompilerParams(
            dimension_semantics=("arbitrary", "arbitrary"),
        ),
    )(batch_address, embt, freq, memory_matrix)
